# baseline (speedup 1.0000x reference)
import functools

import jax
import jax.numpy as jnp
from jax import lax
from jax.experimental import pallas as pl
from jax.experimental.pallas import tpu as pltpu
from jax.experimental.pallas import tpu_sc as plsc

B = 16384
D = 64
H = 64
NC = 2
NS = 16
NW = NC * NS
BPW = B // NW
CHB = 128
NCH = BPW // CHB


def _sc_gather(uid, iid, utab, itab):
    mesh = plsc.VectorSubcoreMesh(core_axis_name="c", subcore_axis_name="s")

    @functools.partial(
        pl.kernel,
        mesh=mesh,
        compiler_params=pltpu.CompilerParams(use_tc_tiling_on_sc=False),
        out_type=(
            jax.ShapeDtypeStruct((B, D), jnp.float32),
            jax.ShapeDtypeStruct((B, D), jnp.float32),
        ),
        scratch_types=[
            pltpu.VMEM((NCH, CHB), jnp.int32),
            pltpu.VMEM((NCH, CHB), jnp.int32),
            pltpu.VMEM((BPW, D), jnp.float32),
            pltpu.VMEM((BPW, D), jnp.float32),
            pltpu.SemaphoreType.DMA,
        ],
    )
    def gk(uid_h, iid_h, utab_h, itab_h, ue_h, ie_h, uix, iix, urows, irows, sem):
        wid = lax.axis_index("s") * NC + lax.axis_index("c")
        base = wid * BPW
        for j in range(NCH):
            pltpu.sync_copy(uid_h.at[pl.ds(base + j * CHB, CHB)], uix.at[j])
            pltpu.sync_copy(iid_h.at[pl.ds(base + j * CHB, CHB)], iix.at[j])
        copies = []
        for j in range(NCH):
            copies.append(
                pltpu.async_copy(utab_h.at[uix.at[j]], urows.at[pl.ds(j * CHB, CHB)], sem)
            )
            copies.append(
                pltpu.async_copy(itab_h.at[iix.at[j]], irows.at[pl.ds(j * CHB, CHB)], sem)
            )
        for c in copies:
            c.wait()
        pltpu.sync_copy(urows, ue_h.at[pl.ds(base, BPW)])
        pltpu.sync_copy(irows, ie_h.at[pl.ds(base, BPW)])

    return gk(uid, iid, utab, itab)


def _mlp_body(ue, ie, w1u, w1i, b1, w2, b2, y):
    h = jnp.dot(ue[...], w1u[...], preferred_element_type=jnp.float32)
    h = h + jnp.dot(ie[...], w1i[...], preferred_element_type=jnp.float32)
    h = jnp.maximum(h + b1[...], 0.0)
    y[...] = jnp.dot(h, w2[...], preferred_element_type=jnp.float32) + b2[0, 0]


def _tc_mlp(ue, ie, w1u, w1i, b1, w2, b2):
    CH = 2048
    grid = (B // CH,)
    return pl.pallas_call(
        _mlp_body,
        grid=grid,
        in_specs=[
            pl.BlockSpec((CH, D), lambda i: (i, 0)),
            pl.BlockSpec((CH, D), lambda i: (i, 0)),
            pl.BlockSpec((D, H), lambda i: (0, 0)),
            pl.BlockSpec((D, H), lambda i: (0, 0)),
            pl.BlockSpec((1, H), lambda i: (0, 0)),
            pl.BlockSpec((H, 1), lambda i: (0, 0)),
            pl.BlockSpec((1, 1), lambda i: (0, 0)),
        ],
        out_specs=pl.BlockSpec((CH, 1), lambda i: (i, 0)),
        out_shape=jax.ShapeDtypeStruct((B, 1), jnp.float32),
    )(ue, ie, w1u, w1i, b1, w2, b2)


def kernel(userID, ItemID, user_table, item_table, W1, b1, W2, b2):
    ue, ie = _sc_gather(userID, ItemID, user_table, item_table)
    w1u = W1[:D]
    w1i = W1[D:]
    y = _tc_mlp(ue, ie, w1u, w1i, b1.reshape(1, H), W2, b2.reshape(1, 1))
    return jnp.squeeze(y, axis=-1)


# speedup vs baseline: 1.1986x; 1.1986x over previous
import functools

import jax
import jax.numpy as jnp
from jax import lax
from jax.experimental import pallas as pl
from jax.experimental.pallas import tpu as pltpu
from jax.experimental.pallas import tpu_sc as plsc

B = 16384
V = 1000000
D = 64
H = 64
NC = 2
NS = 16
NW = NC * NS
BPW = B // NW
CHB = 128
NCH = BPW // CHB

PW = 1024
M0 = 489 * PW
NBLK = (V + PW - 1) // PW
NPB = M0 // PW


def _pack_body(x1, x2, out):
    out[...] = jnp.concatenate(
        [jnp.transpose(x1[...]), jnp.transpose(x2[...])], axis=1
    )


def _tc_pack(tab_t):
    return pl.pallas_call(
        _pack_body,
        grid=(NPB,),
        in_specs=[
            pl.BlockSpec((D, PW), lambda j: (0, j)),
            pl.BlockSpec((D, PW), lambda j: (0, jnp.minimum(j + NPB, NBLK - 1))),
        ],
        out_specs=pl.BlockSpec((PW, 2 * D), lambda j: (j, 0)),
        out_shape=jax.ShapeDtypeStruct((M0, 2 * D), jnp.float32),
    )(tab_t, tab_t)


def _sc_gather(uslot, islot, utp, itp):
    mesh = plsc.VectorSubcoreMesh(core_axis_name="c", subcore_axis_name="s")

    @functools.partial(
        pl.kernel,
        mesh=mesh,
        out_type=(
            jax.ShapeDtypeStruct((B, 2 * D), jnp.float32),
            jax.ShapeDtypeStruct((B, 2 * D), jnp.float32),
        ),
        scratch_types=[
            pltpu.VMEM((BPW,), jnp.int32),
            pltpu.VMEM((BPW,), jnp.int32),
            pltpu.VMEM((2, CHB, 2 * D), jnp.float32),
            pltpu.VMEM((2, CHB, 2 * D), jnp.float32),
            pltpu.SemaphoreType.DMA,
            pltpu.SemaphoreType.DMA,
        ],
    )
    def gk(us_h, is_h, ut_h, it_h, ue_h, ie_h, uix, iix, ubuf, ibuf, s0, s1):
        wid = lax.axis_index("s") * NC + lax.axis_index("c")
        base = wid * BPW
        pltpu.sync_copy(us_h.at[pl.ds(base, BPW)], uix)
        pltpu.sync_copy(is_h.at[pl.ds(base, BPW)], iix)
        sems = (s0, s1)
        copies = [None] * NCH

        def fire(c):
            b = c % 2
            cu = pltpu.async_copy(
                ut_h.at[uix.at[pl.ds(c * CHB, CHB)]], ubuf.at[b], sems[b]
            )
            ci = pltpu.async_copy(
                it_h.at[iix.at[pl.ds(c * CHB, CHB)]], ibuf.at[b], sems[b]
            )
            copies[c] = (cu, ci)

        def drain(c):
            b = c % 2
            cu, ci = copies[c]
            cu.wait()
            ci.wait()
            pltpu.sync_copy(ubuf.at[b], ue_h.at[pl.ds(base + c * CHB, CHB)])
            pltpu.sync_copy(ibuf.at[b], ie_h.at[pl.ds(base + c * CHB, CHB)])

        fire(0)
        fire(1)
        for c in range(NCH):
            drain(c)
            if c + 2 < NCH:
                fire(c + 2)

    return gk(uslot, islot, utp, itp)


def _mlp_body(up, ip, usel, isel, w1u, w1i, b1, w2, b2, y):
    ue = jnp.where(usel[...] != 0, up[:, D:], up[:, :D])
    ie = jnp.where(isel[...] != 0, ip[:, D:], ip[:, :D])
    h = jnp.dot(ue, w1u[...], preferred_element_type=jnp.float32)
    h = h + jnp.dot(ie, w1i[...], preferred_element_type=jnp.float32)
    h = jnp.maximum(h + b1[...], 0.0)
    y[...] = jnp.dot(h, w2[...], preferred_element_type=jnp.float32) + b2[0, 0]


def _tc_mlp(uep, iep, usel, isel, w1u, w1i, b1r, W2, b2r):
    CH = 2048
    return pl.pallas_call(
        _mlp_body,
        grid=(B // CH,),
        in_specs=[
            pl.BlockSpec((CH, 2 * D), lambda i: (i, 0)),
            pl.BlockSpec((CH, 2 * D), lambda i: (i, 0)),
            pl.BlockSpec((CH, 1), lambda i: (i, 0)),
            pl.BlockSpec((CH, 1), lambda i: (i, 0)),
            pl.BlockSpec((D, H), lambda i: (0, 0)),
            pl.BlockSpec((D, H), lambda i: (0, 0)),
            pl.BlockSpec((1, H), lambda i: (0, 0)),
            pl.BlockSpec((H, 1), lambda i: (0, 0)),
            pl.BlockSpec((1, 1), lambda i: (0, 0)),
        ],
        out_specs=pl.BlockSpec((CH, 1), lambda i: (i, 0)),
        out_shape=jax.ShapeDtypeStruct((B, 1), jnp.float32),
    )(uep, iep, usel, isel, w1u, w1i, b1r, W2, b2r)


def kernel(userID, ItemID, user_table, item_table, W1, b1, W2, b2):
    utp = _tc_pack(user_table.T)
    itp = _tc_pack(item_table.T)
    uhi = userID >= M0
    ihi = ItemID >= M0
    uslot = jnp.where(uhi, userID - M0, userID)
    islot = jnp.where(ihi, ItemID - M0, ItemID)
    uep, iep = _sc_gather(uslot, islot, utp, itp)
    y = _tc_mlp(
        uep,
        iep,
        uhi.astype(jnp.int32).reshape(B, 1),
        ihi.astype(jnp.int32).reshape(B, 1),
        W1[:D],
        W1[D:],
        b1.reshape(1, H),
        W2,
        b2.reshape(1, 1),
    )
    return jnp.squeeze(y, axis=-1)


# speedup vs baseline: 1.2996x; 1.0843x over previous
import functools

import jax
import jax.numpy as jnp
from jax import lax
from jax.experimental import pallas as pl
from jax.experimental.pallas import tpu as pltpu
from jax.experimental.pallas import tpu_sc as plsc

B = 16384
V = 1000000
D = 64
H = 64
NC = 2
NS = 16
NW = NC * NS
BPW = B // NW
CHB = 128
NCH = BPW // CHB

PW = 1024
M0 = 489 * PW
NBLK = (V + PW - 1) // PW
NPB = M0 // PW


def _pack_body(x1, x2, eye, out):
    dn = (((0,), (0,)), ((), ()))
    xb1 = x1[...].astype(jnp.bfloat16)
    xb2 = x2[...].astype(jnp.bfloat16)
    a = lax.dot_general(xb1, eye[...], dn, preferred_element_type=jnp.float32)
    b = lax.dot_general(xb2, eye[...], dn, preferred_element_type=jnp.float32)
    ab = jnp.concatenate([a, b], axis=1).astype(jnp.bfloat16)
    out[...] = pltpu.bitcast(ab, jnp.float32)


def _tc_pack(tab_t, eye):
    return pl.pallas_call(
        _pack_body,
        grid=(NPB,),
        compiler_params=pltpu.CompilerParams(fuse_transposed_lhs_in_matmul=True),
        in_specs=[
            pl.BlockSpec((D, PW), lambda j: (0, j)),
            pl.BlockSpec((D, PW), lambda j: (0, jnp.minimum(j + NPB, NBLK - 1))),
            pl.BlockSpec((D, D), lambda j: (0, 0)),
        ],
        out_specs=pl.BlockSpec((PW // 2, 2 * D), lambda j: (j, 0)),
        out_shape=jax.ShapeDtypeStruct((M0 // 2, 2 * D), jnp.float32),
    )(tab_t, tab_t, eye)


def _sc_gather(uslot, islot, utp, itp):
    mesh = plsc.VectorSubcoreMesh(core_axis_name="c", subcore_axis_name="s")

    @functools.partial(
        pl.kernel,
        mesh=mesh,
        out_type=(
            jax.ShapeDtypeStruct((B, 2 * D), jnp.float32),
            jax.ShapeDtypeStruct((B, 2 * D), jnp.float32),
        ),
        scratch_types=[
            pltpu.VMEM((BPW,), jnp.int32),
            pltpu.VMEM((BPW,), jnp.int32),
            pltpu.VMEM((2, CHB, 2 * D), jnp.float32),
            pltpu.VMEM((2, CHB, 2 * D), jnp.float32),
            pltpu.SemaphoreType.DMA,
            pltpu.SemaphoreType.DMA,
        ],
    )
    def gk(us_h, is_h, ut_h, it_h, ue_h, ie_h, uix, iix, ubuf, ibuf, s0, s1):
        wid = lax.axis_index("s") * NC + lax.axis_index("c")
        base = wid * BPW
        pltpu.sync_copy(us_h.at[pl.ds(base, BPW)], uix)
        pltpu.sync_copy(is_h.at[pl.ds(base, BPW)], iix)
        sems = (s0, s1)
        copies = [None] * NCH

        def fire(c):
            b = c % 2
            cu = pltpu.async_copy(
                ut_h.at[uix.at[pl.ds(c * CHB, CHB)]], ubuf.at[b], sems[b]
            )
            ci = pltpu.async_copy(
                it_h.at[iix.at[pl.ds(c * CHB, CHB)]], ibuf.at[b], sems[b]
            )
            copies[c] = (cu, ci)

        def drain(c):
            b = c % 2
            cu, ci = copies[c]
            cu.wait()
            ci.wait()
            pltpu.sync_copy(ubuf.at[b], ue_h.at[pl.ds(base + c * CHB, CHB)])
            pltpu.sync_copy(ibuf.at[b], ie_h.at[pl.ds(base + c * CHB, CHB)])

        fire(0)
        fire(1)
        for c in range(NCH):
            drain(c)
            if c + 2 < NCH:
                fire(c + 2)

    return gk(uslot, islot, utp, itp)


def _unpack(p, par, sel):
    w = lax.bitcast_convert_type(p, jnp.uint32)
    lo = w << 16
    hi = w & jnp.uint32(0xFFFF0000)
    bits = jnp.where(par != 0, hi, lo)
    full = lax.bitcast_convert_type(bits, jnp.float32)
    return jnp.where(sel != 0, full[:, D:], full[:, :D])


def _mlp_body(up, ip, usel, upar, isel, ipar, w1u, w1i, b1, w2, b2, y):
    ue = _unpack(up[...], upar[...], usel[...])
    ie = _unpack(ip[...], ipar[...], isel[...])
    h = jnp.dot(ue, w1u[...], preferred_element_type=jnp.float32)
    h = h + jnp.dot(ie, w1i[...], preferred_element_type=jnp.float32)
    h = jnp.maximum(h + b1[...], 0.0)
    y[...] = jnp.dot(h, w2[...], preferred_element_type=jnp.float32) + b2[0, 0]


def _tc_mlp(uep, iep, usel, upar, isel, ipar, w1u, w1i, b1r, W2, b2r):
    CH = 2048
    return pl.pallas_call(
        _mlp_body,
        grid=(B // CH,),
        in_specs=[
            pl.BlockSpec((CH, 2 * D), lambda i: (i, 0)),
            pl.BlockSpec((CH, 2 * D), lambda i: (i, 0)),
            pl.BlockSpec((CH, 1), lambda i: (i, 0)),
            pl.BlockSpec((CH, 1), lambda i: (i, 0)),
            pl.BlockSpec((CH, 1), lambda i: (i, 0)),
            pl.BlockSpec((CH, 1), lambda i: (i, 0)),
            pl.BlockSpec((D, H), lambda i: (0, 0)),
            pl.BlockSpec((D, H), lambda i: (0, 0)),
            pl.BlockSpec((1, H), lambda i: (0, 0)),
            pl.BlockSpec((H, 1), lambda i: (0, 0)),
            pl.BlockSpec((1, 1), lambda i: (0, 0)),
        ],
        out_specs=pl.BlockSpec((CH, 1), lambda i: (i, 0)),
        out_shape=jax.ShapeDtypeStruct((B, 1), jnp.float32),
    )(uep, iep, usel, upar, isel, ipar, w1u, w1i, b1r, W2, b2r)


_PAIR_INTERLEAVED = True


def _slot_par(row):
    if _PAIR_INTERLEAVED:
        return row >> 1, row & 1
    j = row // PW
    p = row % PW
    half = PW // 2
    return j * half + p % half, p // half


def kernel(userID, ItemID, user_table, item_table, W1, b1, W2, b2):
    eye = jnp.eye(D, dtype=jnp.bfloat16)
    utp = _tc_pack(user_table.T, eye)
    itp = _tc_pack(item_table.T, eye)
    uhi = userID >= M0
    ihi = ItemID >= M0
    urow = jnp.where(uhi, userID - M0, userID)
    irow = jnp.where(ihi, ItemID - M0, ItemID)
    uslot, upar = _slot_par(urow)
    islot, ipar = _slot_par(irow)
    uep, iep = _sc_gather(uslot, islot, utp, itp)
    y = _tc_mlp(
        uep,
        iep,
        uhi.astype(jnp.int32).reshape(B, 1),
        upar.reshape(B, 1),
        ihi.astype(jnp.int32).reshape(B, 1),
        ipar.reshape(B, 1),
        W1[:D],
        W1[D:],
        b1.reshape(1, H),
        W2,
        b2.reshape(1, 1),
    )
    return jnp.squeeze(y, axis=-1)


# speedup vs baseline: 2.4014x; 1.8477x over previous
import functools

import jax
import jax.numpy as jnp
from jax import lax
from jax.experimental import pallas as pl
from jax.experimental.pallas import tpu as pltpu
from jax.experimental.pallas import tpu_sc as plsc

B = 16384
V = 1000000
D = 64
H = 64
NC = 2
NS = 16
NW = NC * NS
BPW = B // NW
CHB = 128
NCH = BPW // CHB

PW = 4096
M0 = 123 * PW
NBLK = (V + PW - 1) // PW
NPB = M0 // PW


def _pack_body(x1, x2, eye, out):
    dn = (((0,), (0,)), ((), ()))
    xb1 = x1[...].astype(jnp.bfloat16)
    xb2 = x2[...].astype(jnp.bfloat16)
    a = lax.dot_general(xb1, eye[...], dn, preferred_element_type=jnp.float32)
    b = lax.dot_general(xb2, eye[...], dn, preferred_element_type=jnp.float32)
    ab = jnp.concatenate([a, b], axis=1).astype(jnp.bfloat16)
    out[...] = pltpu.bitcast(ab, jnp.float32)


def _tc_pack(tab_t, eye):
    return pl.pallas_call(
        _pack_body,
        grid=(NPB,),
        compiler_params=pltpu.CompilerParams(fuse_transposed_lhs_in_matmul=True),
        in_specs=[
            pl.BlockSpec((D, PW), lambda j: (0, j)),
            pl.BlockSpec((D, PW), lambda j: (0, jnp.minimum(j + NPB, NBLK - 1))),
            pl.BlockSpec((D, D), lambda j: (0, 0)),
        ],
        out_specs=pl.BlockSpec((PW // 2, 2 * D), lambda j: (j, 0)),
        out_shape=jax.ShapeDtypeStruct((M0 // 2, 2 * D), jnp.float32),
    )(tab_t, tab_t, eye)


def _sc_gather(uslot, islot, utp, itp):
    mesh = plsc.VectorSubcoreMesh(core_axis_name="c", subcore_axis_name="s")

    @functools.partial(
        pl.kernel,
        mesh=mesh,
        out_type=(
            jax.ShapeDtypeStruct((B, 2 * D), jnp.float32),
            jax.ShapeDtypeStruct((B, 2 * D), jnp.float32),
        ),
        scratch_types=[
            pltpu.VMEM((BPW,), jnp.int32),
            pltpu.VMEM((BPW,), jnp.int32),
            pltpu.VMEM((2, CHB, 2 * D), jnp.float32),
            pltpu.VMEM((2, CHB, 2 * D), jnp.float32),
            pltpu.SemaphoreType.DMA,
            pltpu.SemaphoreType.DMA,
        ],
    )
    def gk(us_h, is_h, ut_h, it_h, ue_h, ie_h, uix, iix, ubuf, ibuf, s0, s1):
        wid = lax.axis_index("s") * NC + lax.axis_index("c")
        base = wid * BPW
        pltpu.sync_copy(us_h.at[pl.ds(base, BPW)], uix)
        pltpu.sync_copy(is_h.at[pl.ds(base, BPW)], iix)
        sems = (s0, s1)
        copies = [None] * NCH

        def fire(c):
            b = c % 2
            cu = pltpu.async_copy(
                ut_h.at[uix.at[pl.ds(c * CHB, CHB)]], ubuf.at[b], sems[b]
            )
            ci = pltpu.async_copy(
                it_h.at[iix.at[pl.ds(c * CHB, CHB)]], ibuf.at[b], sems[b]
            )
            copies[c] = (cu, ci)

        def drain(c):
            b = c % 2
            cu, ci = copies[c]
            cu.wait()
            ci.wait()
            pltpu.sync_copy(ubuf.at[b], ue_h.at[pl.ds(base + c * CHB, CHB)])
            pltpu.sync_copy(ibuf.at[b], ie_h.at[pl.ds(base + c * CHB, CHB)])

        fire(0)
        fire(1)
        for c in range(NCH):
            drain(c)
            if c + 2 < NCH:
                fire(c + 2)

    return gk(uslot, islot, utp, itp)


def _unpack(p, par, sel):
    w = lax.bitcast_convert_type(p, jnp.uint32)
    lo = w << 16
    hi = w & jnp.uint32(0xFFFF0000)
    bits = jnp.where(par != 0, hi, lo)
    full = lax.bitcast_convert_type(bits, jnp.float32)
    return jnp.where(sel != 0, full[:, D:], full[:, :D])


def _mlp_body(up, ip, usel, upar, isel, ipar, w1u, w1i, b1, w2, b2, y):
    ue = _unpack(up[...], upar[...], usel[...])
    ie = _unpack(ip[...], ipar[...], isel[...])
    h = jnp.dot(ue, w1u[...], preferred_element_type=jnp.float32)
    h = h + jnp.dot(ie, w1i[...], preferred_element_type=jnp.float32)
    h = jnp.maximum(h + b1[...], 0.0)
    y[...] = jnp.dot(h, w2[...], preferred_element_type=jnp.float32) + b2[0, 0]


def _tc_mlp(uep, iep, usel, upar, isel, ipar, w1u, w1i, b1r, W2, b2r):
    CH = 2048
    return pl.pallas_call(
        _mlp_body,
        grid=(B // CH,),
        in_specs=[
            pl.BlockSpec((CH, 2 * D), lambda i: (i, 0)),
            pl.BlockSpec((CH, 2 * D), lambda i: (i, 0)),
            pl.BlockSpec((CH, 1), lambda i: (i, 0)),
            pl.BlockSpec((CH, 1), lambda i: (i, 0)),
            pl.BlockSpec((CH, 1), lambda i: (i, 0)),
            pl.BlockSpec((CH, 1), lambda i: (i, 0)),
            pl.BlockSpec((D, H), lambda i: (0, 0)),
            pl.BlockSpec((D, H), lambda i: (0, 0)),
            pl.BlockSpec((1, H), lambda i: (0, 0)),
            pl.BlockSpec((H, 1), lambda i: (0, 0)),
            pl.BlockSpec((1, 1), lambda i: (0, 0)),
        ],
        out_specs=pl.BlockSpec((CH, 1), lambda i: (i, 0)),
        out_shape=jax.ShapeDtypeStruct((B, 1), jnp.float32),
    )(uep, iep, usel, upar, isel, ipar, w1u, w1i, b1r, W2, b2r)


_PAIR_INTERLEAVED = True


def _slot_par(row):
    if _PAIR_INTERLEAVED:
        return row >> 1, row & 1
    j = row // PW
    p = row % PW
    half = PW // 2
    return j * half + p % half, p // half


def kernel(userID, ItemID, user_table, item_table, W1, b1, W2, b2):
    eye = jnp.eye(D, dtype=jnp.bfloat16)
    utp = _tc_pack(user_table.T, eye)
    itp = _tc_pack(item_table.T, eye)
    uhi = userID >= M0
    ihi = ItemID >= M0
    urow = jnp.where(uhi, userID - M0, userID)
    irow = jnp.where(ihi, ItemID - M0, ItemID)
    uslot, upar = _slot_par(urow)
    islot, ipar = _slot_par(irow)
    uep, iep = _sc_gather(uslot, islot, utp, itp)
    y = _tc_mlp(
        uep,
        iep,
        uhi.astype(jnp.int32).reshape(B, 1),
        upar.reshape(B, 1),
        ihi.astype(jnp.int32).reshape(B, 1),
        ipar.reshape(B, 1),
        W1[:D],
        W1[D:],
        b1.reshape(1, H),
        W2,
        b2.reshape(1, 1),
    )
    return jnp.squeeze(y, axis=-1)


# speedup vs baseline: 2.9053x; 1.2098x over previous
import functools

import jax
import jax.numpy as jnp
from jax import lax
from jax.experimental import pallas as pl
from jax.experimental.pallas import tpu as pltpu
from jax.experimental.pallas import tpu_sc as plsc

B = 16384
V = 1000000
D = 64
H = 64
NC = 2
NS = 16
NW = NC * NS
BPW = B // NW
CHB = 128
NCH = BPW // CHB

PW = 8192
M0 = 62 * PW
NBLK = (V + PW - 1) // PW
NPB = M0 // PW


def _pack_body(x1, x2, eye, out):
    dn = (((0,), (0,)), ((), ()))
    xb1 = x1[...].astype(jnp.bfloat16)
    xb2 = x2[...].astype(jnp.bfloat16)
    a = lax.dot_general(xb1, eye[...], dn, preferred_element_type=jnp.float32)
    b = lax.dot_general(xb2, eye[...], dn, preferred_element_type=jnp.float32)
    ab = jnp.concatenate([a, b], axis=1).astype(jnp.bfloat16)
    out[...] = pltpu.bitcast(ab, jnp.float32)


def _tc_pack(tab_t, eye):
    return pl.pallas_call(
        _pack_body,
        grid=(NPB,),
        compiler_params=pltpu.CompilerParams(fuse_transposed_lhs_in_matmul=True),
        in_specs=[
            pl.BlockSpec((D, PW), lambda j: (0, j)),
            pl.BlockSpec((D, PW), lambda j: (0, jnp.minimum(j + NPB, NBLK - 1))),
            pl.BlockSpec((D, D), lambda j: (0, 0)),
        ],
        out_specs=pl.BlockSpec((PW // 2, 2 * D), lambda j: (j, 0)),
        out_shape=jax.ShapeDtypeStruct((M0 // 2, 2 * D), jnp.float32),
    )(tab_t, tab_t, eye)


def _sc_gather(uslot, islot, utp, itp):
    mesh = plsc.VectorSubcoreMesh(core_axis_name="c", subcore_axis_name="s")

    @functools.partial(
        pl.kernel,
        mesh=mesh,
        out_type=(
            jax.ShapeDtypeStruct((B, 2 * D), jnp.float32),
            jax.ShapeDtypeStruct((B, 2 * D), jnp.float32),
        ),
        scratch_types=[
            pltpu.VMEM((BPW,), jnp.int32),
            pltpu.VMEM((BPW,), jnp.int32),
            pltpu.VMEM((2, CHB, 2 * D), jnp.float32),
            pltpu.VMEM((2, CHB, 2 * D), jnp.float32),
            pltpu.SemaphoreType.DMA,
            pltpu.SemaphoreType.DMA,
        ],
    )
    def gk(us_h, is_h, ut_h, it_h, ue_h, ie_h, uix, iix, ubuf, ibuf, s0, s1):
        wid = lax.axis_index("s") * NC + lax.axis_index("c")
        base = wid * BPW
        pltpu.sync_copy(us_h.at[pl.ds(base, BPW)], uix)
        pltpu.sync_copy(is_h.at[pl.ds(base, BPW)], iix)
        sems = (s0, s1)
        copies = [None] * NCH

        def fire(c):
            b = c % 2
            cu = pltpu.async_copy(
                ut_h.at[uix.at[pl.ds(c * CHB, CHB)]], ubuf.at[b], sems[b]
            )
            ci = pltpu.async_copy(
                it_h.at[iix.at[pl.ds(c * CHB, CHB)]], ibuf.at[b], sems[b]
            )
            copies[c] = (cu, ci)

        def drain(c):
            b = c % 2
            cu, ci = copies[c]
            cu.wait()
            ci.wait()
            pltpu.sync_copy(ubuf.at[b], ue_h.at[pl.ds(base + c * CHB, CHB)])
            pltpu.sync_copy(ibuf.at[b], ie_h.at[pl.ds(base + c * CHB, CHB)])

        fire(0)
        fire(1)
        for c in range(NCH):
            drain(c)
            if c + 2 < NCH:
                fire(c + 2)

    return gk(uslot, islot, utp, itp)


def _unpack(p, par, sel):
    w = lax.bitcast_convert_type(p, jnp.uint32)
    lo = w << 16
    hi = w & jnp.uint32(0xFFFF0000)
    bits = jnp.where(par != 0, hi, lo)
    full = lax.bitcast_convert_type(bits, jnp.float32)
    return jnp.where(sel != 0, full[:, D:], full[:, :D])


def _mlp_body(up, ip, usel, upar, isel, ipar, w1u, w1i, b1, w2, b2, y):
    ue = _unpack(up[...], upar[...], usel[...])
    ie = _unpack(ip[...], ipar[...], isel[...])
    h = jnp.dot(ue, w1u[...], preferred_element_type=jnp.float32)
    h = h + jnp.dot(ie, w1i[...], preferred_element_type=jnp.float32)
    h = jnp.maximum(h + b1[...], 0.0)
    y[...] = jnp.dot(h, w2[...], preferred_element_type=jnp.float32) + b2[0, 0]


def _tc_mlp(uep, iep, usel, upar, isel, ipar, w1u, w1i, b1r, W2, b2r):
    CH = 2048
    return pl.pallas_call(
        _mlp_body,
        grid=(B // CH,),
        in_specs=[
            pl.BlockSpec((CH, 2 * D), lambda i: (i, 0)),
            pl.BlockSpec((CH, 2 * D), lambda i: (i, 0)),
            pl.BlockSpec((CH, 1), lambda i: (i, 0)),
            pl.BlockSpec((CH, 1), lambda i: (i, 0)),
            pl.BlockSpec((CH, 1), lambda i: (i, 0)),
            pl.BlockSpec((CH, 1), lambda i: (i, 0)),
            pl.BlockSpec((D, H), lambda i: (0, 0)),
            pl.BlockSpec((D, H), lambda i: (0, 0)),
            pl.BlockSpec((1, H), lambda i: (0, 0)),
            pl.BlockSpec((H, 1), lambda i: (0, 0)),
            pl.BlockSpec((1, 1), lambda i: (0, 0)),
        ],
        out_specs=pl.BlockSpec((CH, 1), lambda i: (i, 0)),
        out_shape=jax.ShapeDtypeStruct((B, 1), jnp.float32),
    )(uep, iep, usel, upar, isel, ipar, w1u, w1i, b1r, W2, b2r)


_PAIR_INTERLEAVED = True


def _slot_par(row):
    if _PAIR_INTERLEAVED:
        return row >> 1, row & 1
    j = row // PW
    p = row % PW
    half = PW // 2
    return j * half + p % half, p // half


def kernel(userID, ItemID, user_table, item_table, W1, b1, W2, b2):
    eye = jnp.eye(D, dtype=jnp.bfloat16)
    utp = _tc_pack(user_table.T, eye)
    itp = _tc_pack(item_table.T, eye)
    uhi = userID >= M0
    ihi = ItemID >= M0
    urow = jnp.where(uhi, userID - M0, userID)
    irow = jnp.where(ihi, ItemID - M0, ItemID)
    uslot, upar = _slot_par(urow)
    islot, ipar = _slot_par(irow)
    uep, iep = _sc_gather(uslot, islot, utp, itp)
    y = _tc_mlp(
        uep,
        iep,
        uhi.astype(jnp.int32).reshape(B, 1),
        upar.reshape(B, 1),
        ihi.astype(jnp.int32).reshape(B, 1),
        ipar.reshape(B, 1),
        W1[:D],
        W1[D:],
        b1.reshape(1, H),
        W2,
        b2.reshape(1, 1),
    )
    return jnp.squeeze(y, axis=-1)


# speedup vs baseline: 3.2708x; 1.1258x over previous
import functools

import jax
import jax.numpy as jnp
from jax import lax
from jax.experimental import pallas as pl
from jax.experimental.pallas import tpu as pltpu
from jax.experimental.pallas import tpu_sc as plsc

B = 16384
V = 1000000
D = 64
H = 64
NC = 2
NS = 16
NW = NC * NS
BPW = B // NW
CHB = 128
NCH = BPW // CHB

PW = 16384
M0 = 31 * PW
NBLK = (V + PW - 1) // PW
NPB = M0 // PW


def _pack_body(x1, x2, eye, out):
    dn = (((0,), (0,)), ((), ()))
    xb1 = x1[...].astype(jnp.bfloat16)
    xb2 = x2[...].astype(jnp.bfloat16)
    a = lax.dot_general(xb1, eye[...], dn, preferred_element_type=jnp.float32)
    b = lax.dot_general(xb2, eye[...], dn, preferred_element_type=jnp.float32)
    ab = jnp.concatenate([a, b], axis=1).astype(jnp.bfloat16)
    out[...] = pltpu.bitcast(ab, jnp.float32)


def _tc_pack(tab_t, eye):
    return pl.pallas_call(
        _pack_body,
        grid=(NPB,),
        compiler_params=pltpu.CompilerParams(fuse_transposed_lhs_in_matmul=True),
        in_specs=[
            pl.BlockSpec((D, PW), lambda j: (0, j)),
            pl.BlockSpec((D, PW), lambda j: (0, jnp.minimum(j + NPB, NBLK - 1))),
            pl.BlockSpec((D, D), lambda j: (0, 0)),
        ],
        out_specs=pl.BlockSpec((PW // 2, 2 * D), lambda j: (j, 0)),
        out_shape=jax.ShapeDtypeStruct((M0 // 2, 2 * D), jnp.float32),
    )(tab_t, tab_t, eye)


def _sc_gather(uslot, islot, utp, itp):
    mesh = plsc.VectorSubcoreMesh(core_axis_name="c", subcore_axis_name="s")

    @functools.partial(
        pl.kernel,
        mesh=mesh,
        out_type=(
            jax.ShapeDtypeStruct((B, 2 * D), jnp.float32),
            jax.ShapeDtypeStruct((B, 2 * D), jnp.float32),
        ),
        scratch_types=[
            pltpu.VMEM((BPW,), jnp.int32),
            pltpu.VMEM((BPW,), jnp.int32),
            pltpu.VMEM((2, CHB, 2 * D), jnp.float32),
            pltpu.VMEM((2, CHB, 2 * D), jnp.float32),
            pltpu.SemaphoreType.DMA,
            pltpu.SemaphoreType.DMA,
        ],
    )
    def gk(us_h, is_h, ut_h, it_h, ue_h, ie_h, uix, iix, ubuf, ibuf, s0, s1):
        wid = lax.axis_index("s") * NC + lax.axis_index("c")
        base = wid * BPW
        pltpu.sync_copy(us_h.at[pl.ds(base, BPW)], uix)
        pltpu.sync_copy(is_h.at[pl.ds(base, BPW)], iix)
        sems = (s0, s1)
        copies = [None] * NCH

        def fire(c):
            b = c % 2
            cu = pltpu.async_copy(
                ut_h.at[uix.at[pl.ds(c * CHB, CHB)]], ubuf.at[b], sems[b]
            )
            ci = pltpu.async_copy(
                it_h.at[iix.at[pl.ds(c * CHB, CHB)]], ibuf.at[b], sems[b]
            )
            copies[c] = (cu, ci)

        def drain(c):
            b = c % 2
            cu, ci = copies[c]
            cu.wait()
            ci.wait()
            pltpu.sync_copy(ubuf.at[b], ue_h.at[pl.ds(base + c * CHB, CHB)])
            pltpu.sync_copy(ibuf.at[b], ie_h.at[pl.ds(base + c * CHB, CHB)])

        fire(0)
        fire(1)
        for c in range(NCH):
            drain(c)
            if c + 2 < NCH:
                fire(c + 2)

    return gk(uslot, islot, utp, itp)


def _unpack(p, par, sel):
    w = lax.bitcast_convert_type(p, jnp.uint32)
    lo = w << 16
    hi = w & jnp.uint32(0xFFFF0000)
    bits = jnp.where(par != 0, hi, lo)
    full = lax.bitcast_convert_type(bits, jnp.float32)
    return jnp.where(sel != 0, full[:, D:], full[:, :D])


def _mlp_body(up, ip, usel, upar, isel, ipar, w1u, w1i, b1, w2, b2, y):
    ue = _unpack(up[...], upar[...], usel[...])
    ie = _unpack(ip[...], ipar[...], isel[...])
    h = jnp.dot(ue, w1u[...], preferred_element_type=jnp.float32)
    h = h + jnp.dot(ie, w1i[...], preferred_element_type=jnp.float32)
    h = jnp.maximum(h + b1[...], 0.0)
    y[...] = jnp.dot(h, w2[...], preferred_element_type=jnp.float32) + b2[0, 0]


def _tc_mlp(uep, iep, usel, upar, isel, ipar, w1u, w1i, b1r, W2, b2r):
    CH = 2048
    return pl.pallas_call(
        _mlp_body,
        grid=(B // CH,),
        in_specs=[
            pl.BlockSpec((CH, 2 * D), lambda i: (i, 0)),
            pl.BlockSpec((CH, 2 * D), lambda i: (i, 0)),
            pl.BlockSpec((CH, 1), lambda i: (i, 0)),
            pl.BlockSpec((CH, 1), lambda i: (i, 0)),
            pl.BlockSpec((CH, 1), lambda i: (i, 0)),
            pl.BlockSpec((CH, 1), lambda i: (i, 0)),
            pl.BlockSpec((D, H), lambda i: (0, 0)),
            pl.BlockSpec((D, H), lambda i: (0, 0)),
            pl.BlockSpec((1, H), lambda i: (0, 0)),
            pl.BlockSpec((H, 1), lambda i: (0, 0)),
            pl.BlockSpec((1, 1), lambda i: (0, 0)),
        ],
        out_specs=pl.BlockSpec((CH, 1), lambda i: (i, 0)),
        out_shape=jax.ShapeDtypeStruct((B, 1), jnp.float32),
    )(uep, iep, usel, upar, isel, ipar, w1u, w1i, b1r, W2, b2r)


_PAIR_INTERLEAVED = True


def _slot_par(row):
    if _PAIR_INTERLEAVED:
        return row >> 1, row & 1
    j = row // PW
    p = row % PW
    half = PW // 2
    return j * half + p % half, p // half


def kernel(userID, ItemID, user_table, item_table, W1, b1, W2, b2):
    eye = jnp.eye(D, dtype=jnp.bfloat16)
    utp = _tc_pack(user_table.T, eye)
    itp = _tc_pack(item_table.T, eye)
    uhi = userID >= M0
    ihi = ItemID >= M0
    urow = jnp.where(uhi, userID - M0, userID)
    irow = jnp.where(ihi, ItemID - M0, ItemID)
    uslot, upar = _slot_par(urow)
    islot, ipar = _slot_par(irow)
    uep, iep = _sc_gather(uslot, islot, utp, itp)
    y = _tc_mlp(
        uep,
        iep,
        uhi.astype(jnp.int32).reshape(B, 1),
        upar.reshape(B, 1),
        ihi.astype(jnp.int32).reshape(B, 1),
        ipar.reshape(B, 1),
        W1[:D],
        W1[D:],
        b1.reshape(1, H),
        W2,
        b2.reshape(1, 1),
    )
    return jnp.squeeze(y, axis=-1)


# speedup vs baseline: 3.3012x; 1.0093x over previous
import functools

import jax
import jax.numpy as jnp
from jax import lax
from jax.experimental import pallas as pl
from jax.experimental.pallas import tpu as pltpu
from jax.experimental.pallas import tpu_sc as plsc

B = 16384
V = 1000000
D = 64
H = 64
NC = 2
NS = 16
NW = NC * NS
BPW = B // NW
CHB = 128
NCH = BPW // CHB

PW = 32768
NBLK = (V + PW - 1) // PW
SLOTS = NBLK * (PW // 4)


def _pack_body(x, eye, out):
    dn = (((0,), (0,)), ((), ()))
    xb = x[...].astype(jnp.bfloat16)
    a = lax.dot_general(xb, eye[...], dn, preferred_element_type=jnp.float32)
    q = pltpu.bitcast(a.astype(jnp.bfloat16), jnp.float32)
    out[...] = jnp.concatenate([q[: PW // 4], q[PW // 4 :]], axis=1)


def _tc_pack(tab_t, eye):
    return pl.pallas_call(
        _pack_body,
        grid=(NBLK,),
        compiler_params=pltpu.CompilerParams(fuse_transposed_lhs_in_matmul=True),
        in_specs=[
            pl.BlockSpec((D, PW), lambda j: (0, j)),
            pl.BlockSpec((D, D), lambda j: (0, 0)),
        ],
        out_specs=pl.BlockSpec((PW // 4, 2 * D), lambda j: (j, 0)),
        out_shape=jax.ShapeDtypeStruct((SLOTS, 2 * D), jnp.float32),
    )(tab_t, eye)


def _sc_gather(uslot, islot, utp, itp):
    mesh = plsc.VectorSubcoreMesh(core_axis_name="c", subcore_axis_name="s")

    @functools.partial(
        pl.kernel,
        mesh=mesh,
        out_type=(
            jax.ShapeDtypeStruct((B, 2 * D), jnp.float32),
            jax.ShapeDtypeStruct((B, 2 * D), jnp.float32),
        ),
        scratch_types=[
            pltpu.VMEM((BPW,), jnp.int32),
            pltpu.VMEM((BPW,), jnp.int32),
            pltpu.VMEM((2, CHB, 2 * D), jnp.float32),
            pltpu.VMEM((2, CHB, 2 * D), jnp.float32),
            pltpu.SemaphoreType.DMA,
            pltpu.SemaphoreType.DMA,
        ],
    )
    def gk(us_h, is_h, ut_h, it_h, ue_h, ie_h, uix, iix, ubuf, ibuf, s0, s1):
        wid = lax.axis_index("s") * NC + lax.axis_index("c")
        base = wid * BPW
        pltpu.sync_copy(us_h.at[pl.ds(base, BPW)], uix)
        pltpu.sync_copy(is_h.at[pl.ds(base, BPW)], iix)
        sems = (s0, s1)
        copies = [None] * NCH

        def fire(c):
            b = c % 2
            cu = pltpu.async_copy(
                ut_h.at[uix.at[pl.ds(c * CHB, CHB)]], ubuf.at[b], sems[b]
            )
            ci = pltpu.async_copy(
                it_h.at[iix.at[pl.ds(c * CHB, CHB)]], ibuf.at[b], sems[b]
            )
            copies[c] = (cu, ci)

        def drain(c):
            b = c % 2
            cu, ci = copies[c]
            cu.wait()
            ci.wait()
            pltpu.sync_copy(ubuf.at[b], ue_h.at[pl.ds(base + c * CHB, CHB)])
            pltpu.sync_copy(ibuf.at[b], ie_h.at[pl.ds(base + c * CHB, CHB)])

        fire(0)
        fire(1)
        for c in range(NCH):
            drain(c)
            if c + 2 < NCH:
                fire(c + 2)

    return gk(uslot, islot, utp, itp)


def _unpack(p, par, sel):
    w = lax.bitcast_convert_type(p, jnp.uint32)
    lo = w << 16
    hi = w & jnp.uint32(0xFFFF0000)
    bits = jnp.where(par != 0, hi, lo)
    full = lax.bitcast_convert_type(bits, jnp.float32)
    return jnp.where(sel != 0, full[:, D:], full[:, :D])


def _mlp_body(up, ip, usel, upar, isel, ipar, w1u, w1i, b1, w2, b2, y):
    ue = _unpack(up[...], upar[...], usel[...])
    ie = _unpack(ip[...], ipar[...], isel[...])
    h = jnp.dot(ue, w1u[...], preferred_element_type=jnp.float32)
    h = h + jnp.dot(ie, w1i[...], preferred_element_type=jnp.float32)
    h = jnp.maximum(h + b1[...], 0.0)
    y[...] = jnp.dot(h, w2[...], preferred_element_type=jnp.float32) + b2[0, 0]


def _tc_mlp(uep, iep, usel, upar, isel, ipar, w1u, w1i, b1r, W2, b2r):
    CH = 2048
    return pl.pallas_call(
        _mlp_body,
        grid=(B // CH,),
        in_specs=[
            pl.BlockSpec((CH, 2 * D), lambda i: (i, 0)),
            pl.BlockSpec((CH, 2 * D), lambda i: (i, 0)),
            pl.BlockSpec((CH, 1), lambda i: (i, 0)),
            pl.BlockSpec((CH, 1), lambda i: (i, 0)),
            pl.BlockSpec((CH, 1), lambda i: (i, 0)),
            pl.BlockSpec((CH, 1), lambda i: (i, 0)),
            pl.BlockSpec((D, H), lambda i: (0, 0)),
            pl.BlockSpec((D, H), lambda i: (0, 0)),
            pl.BlockSpec((1, H), lambda i: (0, 0)),
            pl.BlockSpec((H, 1), lambda i: (0, 0)),
            pl.BlockSpec((1, 1), lambda i: (0, 0)),
        ],
        out_specs=pl.BlockSpec((CH, 1), lambda i: (i, 0)),
        out_shape=jax.ShapeDtypeStruct((B, 1), jnp.float32),
    )(uep, iep, usel, upar, isel, ipar, w1u, w1i, b1r, W2, b2r)


def _slot_sel_par(i):
    w = i & (PW - 1)
    slot = (i // PW) * (PW // 4) + ((w & (PW // 2 - 1)) >> 1)
    sel = (w >= PW // 2).astype(jnp.int32)
    par = i & 1
    return slot, sel, par


def kernel(userID, ItemID, user_table, item_table, W1, b1, W2, b2):
    eye = jnp.eye(D, dtype=jnp.bfloat16)
    utp = _tc_pack(user_table.T, eye)
    itp = _tc_pack(item_table.T, eye)
    uslot, uhi, upar = _slot_sel_par(userID)
    islot, ihi, ipar = _slot_sel_par(ItemID)
    uep, iep = _sc_gather(uslot, islot, utp, itp)
    y = _tc_mlp(
        uep,
        iep,
        uhi.astype(jnp.int32).reshape(B, 1),
        upar.reshape(B, 1),
        ihi.astype(jnp.int32).reshape(B, 1),
        ipar.reshape(B, 1),
        W1[:D],
        W1[D:],
        b1.reshape(1, H),
        W2,
        b2.reshape(1, 1),
    )
    return jnp.squeeze(y, axis=-1)


# speedup vs baseline: 3.3618x; 1.0184x over previous
import functools

import jax
import jax.numpy as jnp
from jax import lax
from jax.experimental import pallas as pl
from jax.experimental.pallas import tpu as pltpu
from jax.experimental.pallas import tpu_sc as plsc

B = 16384
V = 1000000
D = 64
H = 64
NC = 2
NS = 16
NW = NC * NS
BPW = B // NW
CHB = 128
NCH = BPW // CHB

PW = 49152
NBLK = (V + PW - 1) // PW
SLOTS = NBLK * (PW // 4)


def _pack_body(x, eye, out):
    dn = (((0,), (0,)), ((), ()))
    xb = x[...].astype(jnp.bfloat16)
    a = lax.dot_general(xb, eye[...], dn, preferred_element_type=jnp.float32)
    q = pltpu.bitcast(a.astype(jnp.bfloat16), jnp.float32)
    out[...] = jnp.concatenate([q[: PW // 4], q[PW // 4 :]], axis=1)


def _tc_pack(tab_t, eye):
    return pl.pallas_call(
        _pack_body,
        grid=(NBLK,),
        compiler_params=pltpu.CompilerParams(fuse_transposed_lhs_in_matmul=True),
        in_specs=[
            pl.BlockSpec((D, PW), lambda j: (0, j)),
            pl.BlockSpec((D, D), lambda j: (0, 0)),
        ],
        out_specs=pl.BlockSpec((PW // 4, 2 * D), lambda j: (j, 0)),
        out_shape=jax.ShapeDtypeStruct((SLOTS, 2 * D), jnp.float32),
    )(tab_t, eye)


def _sc_gather(uslot, islot, utp, itp):
    mesh = plsc.VectorSubcoreMesh(core_axis_name="c", subcore_axis_name="s")

    @functools.partial(
        pl.kernel,
        mesh=mesh,
        out_type=(
            jax.ShapeDtypeStruct((B, 2 * D), jnp.float32),
            jax.ShapeDtypeStruct((B, 2 * D), jnp.float32),
        ),
        scratch_types=[
            pltpu.VMEM((BPW,), jnp.int32),
            pltpu.VMEM((BPW,), jnp.int32),
            pltpu.VMEM((2, CHB, 2 * D), jnp.float32),
            pltpu.VMEM((2, CHB, 2 * D), jnp.float32),
            pltpu.SemaphoreType.DMA,
            pltpu.SemaphoreType.DMA,
        ],
    )
    def gk(us_h, is_h, ut_h, it_h, ue_h, ie_h, uix, iix, ubuf, ibuf, s0, s1):
        wid = lax.axis_index("s") * NC + lax.axis_index("c")
        base = wid * BPW
        pltpu.sync_copy(us_h.at[pl.ds(base, BPW)], uix)
        pltpu.sync_copy(is_h.at[pl.ds(base, BPW)], iix)
        sems = (s0, s1)
        copies = [None] * NCH

        def fire(c):
            b = c % 2
            cu = pltpu.async_copy(
                ut_h.at[uix.at[pl.ds(c * CHB, CHB)]], ubuf.at[b], sems[b]
            )
            ci = pltpu.async_copy(
                it_h.at[iix.at[pl.ds(c * CHB, CHB)]], ibuf.at[b], sems[b]
            )
            copies[c] = (cu, ci)

        def drain(c):
            b = c % 2
            cu, ci = copies[c]
            cu.wait()
            ci.wait()
            pltpu.sync_copy(ubuf.at[b], ue_h.at[pl.ds(base + c * CHB, CHB)])
            pltpu.sync_copy(ibuf.at[b], ie_h.at[pl.ds(base + c * CHB, CHB)])

        fire(0)
        fire(1)
        for c in range(NCH):
            drain(c)
            if c + 2 < NCH:
                fire(c + 2)

    return gk(uslot, islot, utp, itp)


def _unpack(p, par, sel):
    w = lax.bitcast_convert_type(p, jnp.uint32)
    lo = w << 16
    hi = w & jnp.uint32(0xFFFF0000)
    bits = jnp.where(par != 0, hi, lo)
    full = lax.bitcast_convert_type(bits, jnp.float32)
    return jnp.where(sel != 0, full[:, D:], full[:, :D])


def _mlp_body(up, ip, usel, upar, isel, ipar, w1u, w1i, b1, w2, b2, y):
    ue = _unpack(up[...], upar[...], usel[...])
    ie = _unpack(ip[...], ipar[...], isel[...])
    h = jnp.dot(ue, w1u[...], preferred_element_type=jnp.float32)
    h = h + jnp.dot(ie, w1i[...], preferred_element_type=jnp.float32)
    h = jnp.maximum(h + b1[...], 0.0)
    y[...] = jnp.dot(h, w2[...], preferred_element_type=jnp.float32) + b2[0, 0]


def _tc_mlp(uep, iep, usel, upar, isel, ipar, w1u, w1i, b1r, W2, b2r):
    CH = 2048
    return pl.pallas_call(
        _mlp_body,
        grid=(B // CH,),
        in_specs=[
            pl.BlockSpec((CH, 2 * D), lambda i: (i, 0)),
            pl.BlockSpec((CH, 2 * D), lambda i: (i, 0)),
            pl.BlockSpec((CH, 1), lambda i: (i, 0)),
            pl.BlockSpec((CH, 1), lambda i: (i, 0)),
            pl.BlockSpec((CH, 1), lambda i: (i, 0)),
            pl.BlockSpec((CH, 1), lambda i: (i, 0)),
            pl.BlockSpec((D, H), lambda i: (0, 0)),
            pl.BlockSpec((D, H), lambda i: (0, 0)),
            pl.BlockSpec((1, H), lambda i: (0, 0)),
            pl.BlockSpec((H, 1), lambda i: (0, 0)),
            pl.BlockSpec((1, 1), lambda i: (0, 0)),
        ],
        out_specs=pl.BlockSpec((CH, 1), lambda i: (i, 0)),
        out_shape=jax.ShapeDtypeStruct((B, 1), jnp.float32),
    )(uep, iep, usel, upar, isel, ipar, w1u, w1i, b1r, W2, b2r)


def _slot_sel_par(i):
    w = i % PW
    slot = (i // PW) * (PW // 4) + (w % (PW // 2)) // 2
    sel = (w >= PW // 2).astype(jnp.int32)
    par = i & 1
    return slot, sel, par


def kernel(userID, ItemID, user_table, item_table, W1, b1, W2, b2):
    eye = jnp.eye(D, dtype=jnp.bfloat16)
    utp = _tc_pack(user_table.T, eye)
    itp = _tc_pack(item_table.T, eye)
    uslot, uhi, upar = _slot_sel_par(userID)
    islot, ihi, ipar = _slot_sel_par(ItemID)
    uep, iep = _sc_gather(uslot, islot, utp, itp)
    y = _tc_mlp(
        uep,
        iep,
        uhi.astype(jnp.int32).reshape(B, 1),
        upar.reshape(B, 1),
        ihi.astype(jnp.int32).reshape(B, 1),
        ipar.reshape(B, 1),
        W1[:D],
        W1[D:],
        b1.reshape(1, H),
        W2,
        b2.reshape(1, 1),
    )
    return jnp.squeeze(y, axis=-1)
